# baseline (speedup 1.0000x reference)
import functools

import jax
import jax.numpy as jnp
from jax import lax
from jax.experimental import pallas as pl
from jax.experimental.pallas import tpu as pltpu
from jax.experimental.pallas import tpu_sc as plsc

N = 10000
E = 320000
F = 128
H = 64

NC = 2
NS = 16
NW = NC * NS

NP = 10240
CHUNK = 128
NBC = E // CHUNK
NBW = NBC // NW
NEXTRA = NBC - NBW * NW
NBUF = 8
ROWS_PER_TILE = NP // NS

_MESH = plsc.VectorSubcoreMesh(
    core_axis_name="c", subcore_axis_name="s", num_cores=NC, num_subcores=NS
)


def _hist_body(col_hbm, zz_hbm, deg_out, cidx_v, ones_v, acc):
    cid = lax.axis_index("c")
    sid = lax.axis_index("s")
    wid = sid * NC + cid
    for l in range(CHUNK // 16):
        ones_v[pl.ds(l * 16, 16)] = jnp.ones((16,), jnp.float32)
    pltpu.sync_copy(
        zz_hbm.at[pl.ds(sid * ROWS_PER_TILE, ROWS_PER_TILE)],
        acc.at[pl.ds(sid * ROWS_PER_TILE, ROWS_PER_TILE)],
    )
    pltpu.sync_copy(col_hbm.at[pl.ds(wid * NBW, NBW)],
                    cidx_v.at[pl.ds(0, NBW)])

    @pl.when(wid < NEXTRA)
    def _():
        pltpu.sync_copy(col_hbm.at[pl.ds(NW * NBW + wid, 1)],
                        cidx_v.at[pl.ds(NBW, 1)])

    plsc.subcore_barrier()

    def body(j, carry):
        pltpu.sync_copy(ones_v, acc.at[cidx_v.at[j]], add=True)
        return carry

    lax.fori_loop(0, NBW, body, 0)

    @pl.when(wid < NEXTRA)
    def _():
        pltpu.sync_copy(ones_v, acc.at[cidx_v.at[NBW]], add=True)

    plsc.subcore_barrier()
    pltpu.sync_copy(
        acc.at[pl.ds(sid * ROWS_PER_TILE, ROWS_PER_TILE)],
        deg_out.at[cid, pl.ds(sid * ROWS_PER_TILE, ROWS_PER_TILE)],
    )


@functools.partial(
    pl.kernel,
    out_type=jax.ShapeDtypeStruct((NC, NP), jnp.float32),
    mesh=_MESH,
    scratch_types=[
        pltpu.VMEM((NBW + 1, CHUNK), jnp.int32),
        pltpu.VMEM((CHUNK,), jnp.float32),
        pltpu.VMEM_SHARED((NP,), jnp.float32),
    ],
    compiler_params=pltpu.CompilerParams(use_tc_tiling_on_sc=False),
    name="gcn_degree_hist",
)
def _sc_degree(col_hbm, zz_hbm, deg_out, cidx_v, ones_v, acc):
    _hist_body(col_hbm, zz_hbm, deg_out, cidx_v, ones_v, acc)


def _agg_body(y_hbm, row_hbm, col_hbm, out_hbm,
              ridx_v, cidx_v, ebuf, sems, acc):
    cid = lax.axis_index("c")
    sid = lax.axis_index("s")
    def zbody(r, carry):
        for l in range(H // 16):
            ebuf[0, r, pl.ds(l * 16, 16)] = jnp.zeros((16,), jnp.float32)
        return carry

    lax.fori_loop(0, CHUNK, zbody, 0)
    for t in range(ROWS_PER_TILE // CHUNK):
        pltpu.sync_copy(
            ebuf.at[0],
            acc.at[pl.ds(sid * ROWS_PER_TILE + t * CHUNK, CHUNK)],
        )
    plsc.subcore_barrier()

    def run(nb):
        for b in range(min(NBUF, nb)):
            pltpu.async_copy(y_hbm.at[ridx_v.at[b]], ebuf.at[b], sems[b])

        def body(i, carry):
            for b in range(NBUF):
                j = i * NBUF + b

                @pl.when(j < nb)
                def _():
                    pltpu.make_async_copy(
                        y_hbm.at[ridx_v.at[j]], ebuf.at[b], sems[b]
                    ).wait()
                    pltpu.sync_copy(ebuf.at[b], acc.at[cidx_v.at[j]],
                                    add=True)
                    nxt = j + NBUF

                    @pl.when(nxt < nb)
                    def _():
                        pltpu.async_copy(
                            y_hbm.at[ridx_v.at[nxt]], ebuf.at[b], sems[b]
                        )

            return carry

        lax.fori_loop(0, (nb + NBUF - 1) // NBUF, body, 0)

    wid = sid * NC + cid
    pltpu.sync_copy(row_hbm.at[pl.ds(wid * NBW, NBW)],
                    ridx_v.at[pl.ds(0, NBW)])
    pltpu.sync_copy(col_hbm.at[pl.ds(wid * NBW, NBW)],
                    cidx_v.at[pl.ds(0, NBW)])

    @pl.when(wid < NEXTRA)
    def _():
        pltpu.sync_copy(row_hbm.at[pl.ds(NW * NBW + wid, 1)],
                        ridx_v.at[pl.ds(NBW, 1)])
        pltpu.sync_copy(col_hbm.at[pl.ds(NW * NBW + wid, 1)],
                        cidx_v.at[pl.ds(NBW, 1)])
        run(NBW + 1)

    @pl.when(wid >= NEXTRA)
    def _():
        run(NBW)

    plsc.subcore_barrier()
    pltpu.sync_copy(
        acc.at[pl.ds(sid * ROWS_PER_TILE, ROWS_PER_TILE)],
        out_hbm.at[cid, pl.ds(sid * ROWS_PER_TILE, ROWS_PER_TILE)],
    )


@functools.partial(
    pl.kernel,
    out_type=jax.ShapeDtypeStruct((NC, NP, H), jnp.float32),
    mesh=_MESH,
    scratch_types=[
        pltpu.VMEM((NBW + 1, CHUNK), jnp.int32),
        pltpu.VMEM((NBW + 1, CHUNK), jnp.int32),
        pltpu.VMEM((NBUF, CHUNK, H), jnp.float32),
        [pltpu.SemaphoreType.DMA] * NBUF,
        pltpu.VMEM_SHARED((NP, H), jnp.float32),
    ],
    compiler_params=pltpu.CompilerParams(use_tc_tiling_on_sc=False),
    name="gcn_edge_agg",
)
def _sc_edge_agg(y_hbm, row_hbm, col_hbm, out_hbm,
                 ridx_v, cidx_v, ebuf, sems, acc):
    _agg_body(y_hbm, row_hbm, col_hbm, out_hbm,
              ridx_v, cidx_v, ebuf, sems, acc)


BL = 1024


def _scale_mm_body(deg_ref, x_ref, w_ref, dinv_out, s_out):
    d = deg_ref[0] + deg_ref[1] + 1.0
    dinv = jnp.where(d > 0, lax.rsqrt(d), 0.0)
    xw = jnp.dot(x_ref[...], w_ref[...], preferred_element_type=jnp.float32)
    dinv_out[...] = dinv
    s_out[...] = xw * dinv


def _mid_body(p_ref, s_ref, dinv_ref, b_ref, w_ref, out_ref):
    dinv = dinv_ref[...]
    h = dinv * (p_ref[0] + p_ref[1] + s_ref[...]) + b_ref[...]
    h = jnp.maximum(h, 0.0)
    out_ref[...] = (
        jnp.dot(h, w_ref[...], preferred_element_type=jnp.float32) * dinv
    )


def _head_body(p_ref, s_ref, dinv_ref, b_ref, wo_ref, bo_ref, out_ref):
    dinv = dinv_ref[...]
    h = dinv * (p_ref[0] + p_ref[1] + s_ref[...]) + b_ref[...]
    h = jnp.maximum(h, 0.0)
    out_ref[...] = (
        jnp.dot(h, wo_ref[...], preferred_element_type=jnp.float32) + bo_ref[...]
    )


def _node_spec(width):
    return pl.BlockSpec((BL, width), lambda i: (i, 0))


def _pair_spec(width):
    return pl.BlockSpec((NC, BL, width), lambda i: (0, i, 0))


def _full_spec(shape):
    return pl.BlockSpec(shape, lambda i: (0,) * len(shape))


def _tc_scale_mm(deg, xp, w1t):
    return pl.pallas_call(
        _scale_mm_body,
        grid=(NP // BL,),
        in_specs=[
            _pair_spec(1),
            _node_spec(F),
            _full_spec((F, H)),
        ],
        out_specs=[_node_spec(1), _node_spec(H)],
        out_shape=[
            jax.ShapeDtypeStruct((NP, 1), jnp.float32),
            jax.ShapeDtypeStruct((NP, H), jnp.float32),
        ],
    )(deg, xp, w1t)


def _tc_mid(p, s, dinv, b, w2t):
    return pl.pallas_call(
        _mid_body,
        grid=(NP // BL,),
        in_specs=[
            _pair_spec(H),
            _node_spec(H),
            _node_spec(1),
            _full_spec((1, H)),
            _full_spec((H, H)),
        ],
        out_specs=_node_spec(H),
        out_shape=jax.ShapeDtypeStruct((NP, H), jnp.float32),
    )(p, s, dinv, b, w2t)


def _tc_head(p, s, dinv, b, wot, bo):
    return pl.pallas_call(
        _head_body,
        grid=(NP // BL,),
        in_specs=[
            _pair_spec(H),
            _node_spec(H),
            _node_spec(1),
            _full_spec((1, H)),
            _full_spec((H, 1)),
            _full_spec((1, 1)),
        ],
        out_specs=_node_spec(1),
        out_shape=jax.ShapeDtypeStruct((NP, 1), jnp.float32),
    )(p, s, dinv, b, wot, bo)


def kernel(x, edge_index, W1, b1, W2, b2, Wo, bo):
    xp = jnp.pad(x, ((0, NP - N), (0, 0)))
    ei = edge_index.astype(jnp.int32)
    row2d = ei[0].reshape(NBC, CHUNK)
    col2d = ei[1].reshape(NBC, CHUNK)
    zz1 = jnp.zeros((NP,), jnp.float32)

    deg = _sc_degree(col2d, zz1)
    dinv, s1 = _tc_scale_mm(deg.reshape(NC, NP, 1), xp, W1.T)
    p1 = _sc_edge_agg(s1, row2d, col2d)
    s2 = _tc_mid(p1, s1, dinv, b1.reshape(1, H), W2.T)
    p2 = _sc_edge_agg(s2, row2d, col2d)
    scores = _tc_head(p2, s2, dinv, b2.reshape(1, H), Wo.T, bo.reshape(1, 1))
    return scores[:N, 0]


# speedup vs baseline: 1.0747x; 1.0747x over previous
import functools

import jax
import jax.numpy as jnp
from jax import lax
from jax.experimental import pallas as pl
from jax.experimental.pallas import tpu as pltpu
from jax.experimental.pallas import tpu_sc as plsc

N = 10000
E = 320000
F = 128
H = 64

NC = 2
NS = 16
NW = NC * NS

NP = 10240
CHUNK = 128
EP = 327680
NBC = EP // CHUNK
NBW = NBC // NW
NBUF = 8
ROWS_PER_TILE = NP // NS

_MESH = plsc.VectorSubcoreMesh(
    core_axis_name="c", subcore_axis_name="s", num_cores=NC, num_subcores=NS
)


def _hist_body(col_hbm, zz_hbm, deg_out, cidx_v, ones_v, acc):
    cid = lax.axis_index("c")
    sid = lax.axis_index("s")
    wid = sid * NC + cid
    for l in range(CHUNK // 16):
        ones_v[pl.ds(l * 16, 16)] = jnp.ones((16,), jnp.float32)
    pltpu.sync_copy(
        zz_hbm.at[pl.ds(sid * ROWS_PER_TILE, ROWS_PER_TILE)],
        acc.at[pl.ds(sid * ROWS_PER_TILE, ROWS_PER_TILE)],
    )
    pltpu.sync_copy(col_hbm.at[pl.ds(wid * NBW, NBW)], cidx_v)
    plsc.subcore_barrier()

    def body(j, carry):
        pltpu.sync_copy(ones_v, acc.at[cidx_v.at[j]], add=True)
        return carry

    lax.fori_loop(0, NBW, body, 0)
    plsc.subcore_barrier()
    pltpu.sync_copy(
        acc.at[pl.ds(sid * ROWS_PER_TILE, ROWS_PER_TILE)],
        deg_out.at[cid, pl.ds(sid * ROWS_PER_TILE, ROWS_PER_TILE)],
    )


@functools.partial(
    pl.kernel,
    out_type=jax.ShapeDtypeStruct((NC, NP), jnp.float32),
    mesh=_MESH,
    scratch_types=[
        pltpu.VMEM((NBW, CHUNK), jnp.int32),
        pltpu.VMEM((CHUNK,), jnp.float32),
        pltpu.VMEM_SHARED((NP,), jnp.float32),
    ],
    compiler_params=pltpu.CompilerParams(use_tc_tiling_on_sc=False),
    name="gcn_degree_hist",
)
def _sc_degree(col_hbm, zz_hbm, deg_out, cidx_v, ones_v, acc):
    _hist_body(col_hbm, zz_hbm, deg_out, cidx_v, ones_v, acc)


def _agg_body(y_hbm, row_hbm, col_hbm, out_hbm,
              ridx_v, cidx_v, ebuf, sems, acc):
    cid = lax.axis_index("c")
    sid = lax.axis_index("s")
    def zbody(r, carry):
        for l in range(H // 16):
            ebuf[0, r, pl.ds(l * 16, 16)] = jnp.zeros((16,), jnp.float32)
        return carry

    lax.fori_loop(0, CHUNK, zbody, 0)
    for t in range(ROWS_PER_TILE // CHUNK):
        pltpu.sync_copy(
            ebuf.at[0],
            acc.at[pl.ds(sid * ROWS_PER_TILE + t * CHUNK, CHUNK)],
        )
    plsc.subcore_barrier()

    wid = sid * NC + cid
    pltpu.sync_copy(row_hbm.at[pl.ds(wid * NBW, NBW)], ridx_v)
    pltpu.sync_copy(col_hbm.at[pl.ds(wid * NBW, NBW)], cidx_v)

    for b in range(NBUF):
        pltpu.async_copy(y_hbm.at[ridx_v.at[b]], ebuf.at[b], sems[b])

    def body(i, carry):
        for b in range(NBUF):
            j = i * NBUF + b
            pltpu.make_async_copy(
                y_hbm.at[ridx_v.at[j]], ebuf.at[b], sems[b]
            ).wait()
            pltpu.sync_copy(ebuf.at[b], acc.at[cidx_v.at[j]], add=True)
            nxt = j + NBUF

            @pl.when(nxt < NBW)
            def _():
                pltpu.async_copy(y_hbm.at[ridx_v.at[nxt]], ebuf.at[b],
                                 sems[b])

        return carry

    lax.fori_loop(0, NBW // NBUF, body, 0)
    plsc.subcore_barrier()
    pltpu.sync_copy(
        acc.at[pl.ds(sid * ROWS_PER_TILE, ROWS_PER_TILE)],
        out_hbm.at[cid, pl.ds(sid * ROWS_PER_TILE, ROWS_PER_TILE)],
    )


@functools.partial(
    pl.kernel,
    out_type=jax.ShapeDtypeStruct((NC, NP, H), jnp.float32),
    mesh=_MESH,
    scratch_types=[
        pltpu.VMEM((NBW, CHUNK), jnp.int32),
        pltpu.VMEM((NBW, CHUNK), jnp.int32),
        pltpu.VMEM((NBUF, CHUNK, H), jnp.float32),
        [pltpu.SemaphoreType.DMA] * NBUF,
        pltpu.VMEM_SHARED((NP, H), jnp.float32),
    ],
    compiler_params=pltpu.CompilerParams(use_tc_tiling_on_sc=False),
    name="gcn_edge_agg",
)
def _sc_edge_agg(y_hbm, row_hbm, col_hbm, out_hbm,
                 ridx_v, cidx_v, ebuf, sems, acc):
    _agg_body(y_hbm, row_hbm, col_hbm, out_hbm,
              ridx_v, cidx_v, ebuf, sems, acc)


BL = 2048


def _scale_mm_body(deg_ref, x_ref, w_ref, dinv_out, s_out):
    d = deg_ref[0] + deg_ref[1] + 1.0
    dinv = jnp.where(d > 0, lax.rsqrt(d), 0.0)
    xw = jnp.dot(x_ref[...], w_ref[...], preferred_element_type=jnp.float32)
    dinv_out[...] = dinv
    s_out[...] = xw * dinv


def _mid_body(p_ref, s_ref, dinv_ref, b_ref, w_ref, out_ref):
    dinv = dinv_ref[...]
    h = dinv * (p_ref[0] + p_ref[1] + s_ref[...]) + b_ref[...]
    h = jnp.maximum(h, 0.0)
    out_ref[...] = (
        jnp.dot(h, w_ref[...], preferred_element_type=jnp.float32) * dinv
    )


def _head_body(p_ref, s_ref, dinv_ref, b_ref, wo_ref, bo_ref, out_ref):
    dinv = dinv_ref[...]
    h = dinv * (p_ref[0] + p_ref[1] + s_ref[...]) + b_ref[...]
    h = jnp.maximum(h, 0.0)
    out_ref[...] = (
        jnp.dot(h, wo_ref[...], preferred_element_type=jnp.float32) + bo_ref[...]
    )


def _node_spec(width):
    return pl.BlockSpec((BL, width), lambda i: (i, 0))


def _pair_spec(width):
    return pl.BlockSpec((NC, BL, width), lambda i: (0, i, 0))


def _full_spec(shape):
    return pl.BlockSpec(shape, lambda i: (0,) * len(shape))


def _tc_scale_mm(deg, xp, w1t):
    return pl.pallas_call(
        _scale_mm_body,
        grid=(NP // BL,),
        in_specs=[
            _pair_spec(1),
            _node_spec(F),
            _full_spec((F, H)),
        ],
        out_specs=[_node_spec(1), _node_spec(H)],
        out_shape=[
            jax.ShapeDtypeStruct((NP, 1), jnp.float32),
            jax.ShapeDtypeStruct((NP, H), jnp.float32),
        ],
    )(deg, xp, w1t)


def _tc_mid(p, s, dinv, b, w2t):
    return pl.pallas_call(
        _mid_body,
        grid=(NP // BL,),
        in_specs=[
            _pair_spec(H),
            _node_spec(H),
            _node_spec(1),
            _full_spec((1, H)),
            _full_spec((H, H)),
        ],
        out_specs=_node_spec(H),
        out_shape=jax.ShapeDtypeStruct((NP, H), jnp.float32),
    )(p, s, dinv, b, w2t)


def _tc_head(p, s, dinv, b, wot, bo):
    return pl.pallas_call(
        _head_body,
        grid=(NP // BL,),
        in_specs=[
            _pair_spec(H),
            _node_spec(H),
            _node_spec(1),
            _full_spec((1, H)),
            _full_spec((H, 1)),
            _full_spec((1, 1)),
        ],
        out_specs=_node_spec(1),
        out_shape=jax.ShapeDtypeStruct((NP, 1), jnp.float32),
    )(p, s, dinv, b, wot, bo)


def kernel(x, edge_index, W1, b1, W2, b2, Wo, bo):
    xp = jnp.pad(x, ((0, NP - N), (0, 0)))
    trash = N + jnp.arange(EP - E, dtype=jnp.int32) % (NP - N)
    pad = jnp.stack([trash, trash])
    ei = jnp.concatenate([edge_index.astype(jnp.int32), pad], axis=1)
    row2d = ei[0].reshape(NBC, CHUNK)
    col2d = ei[1].reshape(NBC, CHUNK)
    zz1 = jnp.zeros((NP,), jnp.float32)

    deg = _sc_degree(col2d, zz1)
    dinv, s1 = _tc_scale_mm(deg.reshape(NC, NP, 1), xp, W1.T)
    p1 = _sc_edge_agg(s1, row2d, col2d)
    s2 = _tc_mid(p1, s1, dinv, b1.reshape(1, H), W2.T)
    p2 = _sc_edge_agg(s2, row2d, col2d)
    scores = _tc_head(p2, s2, dinv, b2.reshape(1, H), Wo.T, bo.reshape(1, 1))
    return scores[:N, 0]
